# Initial kernel scaffold; baseline (speedup 1.0000x reference)
import jax
import jax.numpy as jnp
from jax.experimental import pallas as pl


def kernel(x_graph_1, x_graph_2, edge_index_g1_g1, edge_index_g2_g2, edge_index_g1_g2, edge_index_g2_g1, batch_graph_1, batch_graph_2, slice_dict, params):
    raise NotImplementedError("write your pallas kernel here")


# speedup vs baseline: 1.7015x; 1.7015x over previous
import functools

import jax
import jax.numpy as jnp
from jax import lax
from jax.experimental import pallas as pl
from jax.experimental.pallas import tpu as pltpu

_N = 10000
_H = 128
_BK = 1000
_NG = 16
_NBLK = _N // _BK


def _max_tail(att, amax_ref):
    m = jnp.max(att, axis=0, keepdims=True)

    @pl.when(pl.program_id(0) == 0)
    def _():
        amax_ref[...] = m

    @pl.when(pl.program_id(0) != 0)
    def _():
        amax_ref[...] = jnp.maximum(amax_ref[...], m)


def _pre_body(x_ref, w_ref, b_ref, dinv_ref, va_ref,
              xn_ref, xs_ref, att_ref, amax_ref):
    x = x_ref[...]
    y = jnp.dot(x, w_ref[...], preferred_element_type=jnp.float32) + b_ref[...]
    xn = jnp.maximum(y, 0.0)
    xn_ref[...] = xn
    xs_ref[...] = xn * dinv_ref[...]
    att = jnp.dot(xn, va_ref[...], preferred_element_type=jnp.float32)
    att_ref[...] = att
    _max_tail(att, amax_ref)


def _node_body(aggg_ref, xs_ref, agga_ref, dinv_ref,
               wg_ref, bg_ref, wa_ref, ba_ref, msk_ref, bsk_ref, va_ref,
               xn_ref, xsn_ref, att_ref, amax_ref):
    z = jnp.sum(aggg_ref[...], axis=0) + xs_ref[...]
    t1 = jnp.dot(z, wg_ref[...], preferred_element_type=jnp.float32)
    t1 = t1 * dinv_ref[...] + bg_ref[...]
    a = jnp.sum(agga_ref[...], axis=0)
    t2 = jnp.dot(a, wa_ref[...], preferred_element_type=jnp.float32) + ba_ref[...]
    o = t1 + t2
    xn = jnp.dot(o, msk_ref[...], preferred_element_type=jnp.float32) + bsk_ref[...]
    xn = jnp.maximum(xn, 0.0)
    xn_ref[...] = xn
    xsn_ref[...] = xn * dinv_ref[...]
    att = jnp.dot(xn, va_ref[...], preferred_element_type=jnp.float32)
    att_ref[...] = att
    _max_tail(att, amax_ref)


def _pool_body(x1_ref, b1_ref, x2_ref, b2_ref,
               wp1_ref, bp1_ref, wp2_ref, bp2_ref,
               p1_ref, c1_ref, p2_ref, c2_ref):
    ids = lax.broadcasted_iota(jnp.int32, (1, _NG), 1)

    def one(x_ref, b_ref, wp_ref, bp_ref, p_ref, c_ref):
        xp = jnp.dot(x_ref[...], wp_ref[...],
                     preferred_element_type=jnp.float32) + bp_ref[...]
        oh = (b_ref[...] == ids).astype(jnp.float32)
        ppart = lax.dot_general(oh, xp, (((0,), (0,)), ((), ())),
                                preferred_element_type=jnp.float32)
        cpart = jnp.sum(oh, axis=0, keepdims=True)

        @pl.when(pl.program_id(0) == 0)
        def _():
            p_ref[...] = ppart
            c_ref[...] = cpart

        @pl.when(pl.program_id(0) != 0)
        def _():
            p_ref[...] += ppart
            c_ref[...] += cpart

    one(x1_ref, b1_ref, wp1_ref, bp1_ref, p1_ref, c1_ref)
    one(x2_ref, b2_ref, wp2_ref, bp2_ref, p2_ref, c2_ref)


def _head_body(p1_ref, c1_ref, p2_ref, c2_ref,
               w1_ref, b1_ref, w2_ref, b2_ref, out_ref):
    c1 = jnp.maximum(c1_ref[...], 1.0)
    c2 = jnp.maximum(c2_ref[...], 1.0)
    r1 = p1_ref[...] * (1.0 / c1.T)
    r2 = p2_ref[...] * (1.0 / c2.T)
    r = jnp.concatenate([r1, r2], axis=1)
    h = jnp.dot(r, w1_ref[...], preferred_element_type=jnp.float32) + b1_ref[...]
    h = jnp.maximum(h, 0.0)
    out_ref[...] = jnp.dot(h, w2_ref[...],
                           preferred_element_type=jnp.float32) + b2_ref[...]


def _full(shape):
    return pl.BlockSpec(shape, lambda i: (0,) * len(shape))


def _rows(shape):
    return pl.BlockSpec(shape, lambda i: (i,) + (0,) * (len(shape) - 1))


def _pre_call(x, w, b, dinv, va):
    return pl.pallas_call(
        _pre_body,
        grid=(_NBLK,),
        in_specs=[_rows((_BK, _H)), _full((_H, _H)), _full((1, _H)),
                  _rows((_BK, 1)), _full((_H, 8))],
        out_specs=[_rows((_BK, _H)), _rows((_BK, _H)), _rows((_BK, 8)),
                   _full((1, 8))],
        out_shape=[jax.ShapeDtypeStruct((_N, _H), jnp.float32),
                   jax.ShapeDtypeStruct((_N, _H), jnp.float32),
                   jax.ShapeDtypeStruct((_N, 8), jnp.float32),
                   jax.ShapeDtypeStruct((1, 8), jnp.float32)],
    )(x, w, b, dinv, va)


def _node_call(aggg, xs, agga, dinv, wg, bg, wa, ba, msk, bsk, va):
    np_ = aggg.shape[0]
    na_ = agga.shape[0]
    return pl.pallas_call(
        _node_body,
        grid=(_NBLK,),
        in_specs=[
            pl.BlockSpec((np_, _BK, _H), lambda i: (0, i, 0)),
            _rows((_BK, _H)),
            pl.BlockSpec((na_, _BK, _H), lambda i: (0, i, 0)),
            _rows((_BK, 1)),
            _full((_H, _H)), _full((1, _H)),
            _full((_H, _H)), _full((1, _H)),
            _full((_H, _H)), _full((1, _H)),
            _full((_H, 8)),
        ],
        out_specs=[_rows((_BK, _H)), _rows((_BK, _H)), _rows((_BK, 8)),
                   _full((1, 8))],
        out_shape=[jax.ShapeDtypeStruct((_N, _H), jnp.float32),
                   jax.ShapeDtypeStruct((_N, _H), jnp.float32),
                   jax.ShapeDtypeStruct((_N, 8), jnp.float32),
                   jax.ShapeDtypeStruct((1, 8), jnp.float32)],
    )(aggg, xs, agga, dinv, wg, bg, wa, ba, msk, bsk, va)


def _pool_call(x1, b1, x2, b2, wp1, bp1, wp2, bp2):
    return pl.pallas_call(
        _pool_body,
        grid=(_NBLK,),
        in_specs=[_rows((_BK, _H)), _rows((_BK, 1)),
                  _rows((_BK, _H)), _rows((_BK, 1)),
                  _full((_H, _H)), _full((1, _H)),
                  _full((_H, _H)), _full((1, _H))],
        out_specs=[_full((_NG, _H)), _full((1, _NG)),
                   _full((_NG, _H)), _full((1, _NG))],
        out_shape=[jax.ShapeDtypeStruct((_NG, _H), jnp.float32),
                   jax.ShapeDtypeStruct((1, _NG), jnp.float32),
                   jax.ShapeDtypeStruct((_NG, _H), jnp.float32),
                   jax.ShapeDtypeStruct((1, _NG), jnp.float32)],
    )(x1, b1, x2, b2, wp1, bp1, wp2, bp2)


def _head_call(p1, c1, p2, c2, w1, b1, w2, b2):
    return pl.pallas_call(
        _head_body,
        grid=(1,),
        in_specs=[_full((_NG, _H)), _full((1, _NG)),
                  _full((_NG, _H)), _full((1, _NG)),
                  _full((2 * _H, _H)), _full((1, _H)),
                  _full((_H, 32)), _full((1, 32))],
        out_specs=_full((_NG, 32)),
        out_shape=jax.ShapeDtypeStruct((_NG, 32), jnp.float32),
    )(p1, c1, p2, c2, w1, b1, w2, b2)


def _fold_bn(W, b, g, beta, m, v):
    inv = g * lax.rsqrt(v + 1e-5)
    return (W.T * inv[None, :], ((b - m) * inv + beta)[None, :])


def _lrelu(x):
    return jnp.maximum(x, 0.2 * x)


def _seg_sum_rows(vals, src, dst, coeff, n_dst):
    g = vals[src]
    if coeff is not None:
        g = g * coeff[:, None]
    return jnp.zeros((n_dst, vals.shape[1]), vals.dtype).at[dst].add(g)


def _edge_alpha(als, ald, src, dst, mshift, n_dst):
    e = _lrelu(als[src] + ald[dst])
    ex = jnp.exp(e - mshift)
    den = jnp.zeros((n_dst,), ex.dtype).at[dst].add(ex)
    return ex / jnp.maximum(den[dst], 1e-16)


def kernel(x_graph_1, x_graph_2, edge_index_g1_g1, edge_index_g2_g2,
           edge_index_g1_g2, edge_index_g2_g1, batch_graph_1, batch_graph_2,
           slice_dict, params):
    p = params
    L = 2

    wpre1, bpre1 = _fold_bn(p['pre_W_g1'], p['pre_b_g1'],
                            p['bn_pre_g1_gamma'], p['bn_pre_g1_beta'],
                            p['bn_pre_g1_mean'], p['bn_pre_g1_var'])
    wpre2, bpre2 = _fold_bn(p['pre_W_g2'], p['pre_b_g2'],
                            p['bn_pre_g2_gamma'], p['bn_pre_g2_beta'],
                            p['bn_pre_g2_mean'], p['bn_pre_g2_var'])
    wpost1, bpost1 = _fold_bn(p['post_W_g1'], p['post_b_g1'],
                              p['bn_post_g1_gamma'], p['bn_post_g1_beta'],
                              p['bn_post_g1_mean'], p['bn_post_g1_var'])
    wpost2, bpost2 = _fold_bn(p['post_W_g2'], p['post_b_g2'],
                              p['bn_post_g2_gamma'], p['bn_post_g2_beta'],
                              p['bn_post_g2_mean'], p['bn_post_g2_var'])

    eye = jnp.eye(_H, dtype=jnp.float32)
    zcol = jnp.zeros((_H, 1), jnp.float32)

    def att_w(l):
        v_as12 = p['gat12_Ws_%d' % l].T @ p['gat12_as_%d' % l]
        v_ad12 = p['gat12_Wd_%d' % l].T @ p['gat12_ad_%d' % l]
        v_as21 = p['gat21_Ws_%d' % l].T @ p['gat21_as_%d' % l]
        v_ad21 = p['gat21_Wd_%d' % l].T @ p['gat21_ad_%d' % l]
        va1 = jnp.concatenate([v_as12[:, None], v_ad21[:, None]] + [zcol] * 6,
                              axis=1)
        va2 = jnp.concatenate([v_as21[:, None], v_ad12[:, None]] + [zcol] * 6,
                              axis=1)
        return va1, va2

    va_zero = jnp.zeros((_H, 8), jnp.float32)

    src11, dst11 = edge_index_g1_g1[0], edge_index_g1_g1[1]
    src22, dst22 = edge_index_g2_g2[0], edge_index_g2_g2[1]
    src12, dst12 = edge_index_g1_g2[0], edge_index_g1_g2[1]
    src21, dst21 = edge_index_g2_g1[0], edge_index_g2_g1[1]

    deg1 = jnp.zeros((_N,), jnp.float32).at[dst11].add(1.0) + 1.0
    deg2 = jnp.zeros((_N,), jnp.float32).at[dst22].add(1.0) + 1.0
    dinv1 = lax.rsqrt(deg1)[:, None]
    dinv2 = lax.rsqrt(deg2)[:, None]

    va1, va2 = att_w(0)
    x1, xs1, att1, am1 = _pre_call(x_graph_1, wpre1, bpre1, dinv1, va1)
    x2, xs2, att2, am2 = _pre_call(x_graph_2, wpre2, bpre2, dinv2, va2)

    for l in range(L):
        aggg1 = _seg_sum_rows(xs1, src11, dst11, None, _N)[None]
        aggg2 = _seg_sum_rows(xs2, src22, dst22, None, _N)[None]

        m12 = _lrelu(am1[0, 0] + am2[0, 1])
        m21 = _lrelu(am2[0, 0] + am1[0, 1])
        alpha12 = _edge_alpha(att1[:, 0], att2[:, 1], src12, dst12, m12, _N)
        alpha21 = _edge_alpha(att2[:, 0], att1[:, 1], src21, dst21, m21, _N)
        agga1 = _seg_sum_rows(x2, src21, dst21, alpha21, _N)[None]
        agga2 = _seg_sum_rows(x1, src12, dst12, alpha12, _N)[None]

        van1, van2 = att_w(l + 1) if l + 1 < L else (va_zero, va_zero)
        x1, xs1, att1, am1 = _node_call(
            aggg1, xs1, agga1, dinv1,
            p['gcn11_W_%d' % l].T, p['gcn11_b_%d' % l][None, :],
            p['gat21_Ws_%d' % l].T, p['gat21_b_%d' % l][None, :],
            eye + p['skip_W_g1_%d' % l].T, p['skip_b_g1_%d' % l][None, :],
            van1)
        x2, xs2, att2, am2 = _node_call(
            aggg2, xs2, agga2, dinv2,
            p['gcn22_W_%d' % l].T, p['gcn22_b_%d' % l][None, :],
            p['gat12_Ws_%d' % l].T, p['gat12_b_%d' % l][None, :],
            eye + p['skip_W_g2_%d' % l].T, p['skip_b_g2_%d' % l][None, :],
            van2)

    p1, c1, p2, c2 = _pool_call(
        x1, batch_graph_1[:, None], x2, batch_graph_2[:, None],
        wpost1, bpost1, wpost2, bpost2)
    return _head_call(p1, c1, p2, c2,
                      p['lin1_W'].T, p['lin1_b'][None, :],
                      p['lin2_W'].T, p['lin2_b'][None, :])


# speedup vs baseline: 2.1742x; 1.2778x over previous
import functools

import jax
import jax.numpy as jnp
from jax import lax
from jax.experimental import pallas as pl
from jax.experimental.pallas import tpu as pltpu
from jax.experimental.pallas import tpu_sc as plsc

_N = 10000
_NP = 10240
_H = 128
_BK = 1024
_NG = 16
_NBLK = _NP // _BK

_E = 320000
_EC = 80
_E2D = 4096
_ECR = _E2D // 16
_NPT = _NP // 16
_HH = _H // 2


def _make_sc_agg(with_coeff):
    mesh = plsc.VectorSubcoreMesh(core_axis_name="c", subcore_axis_name="s")
    scratch = [
        pltpu.VMEM((_ECR, _EC), jnp.int32),
        pltpu.VMEM((_ECR, _EC), jnp.int32),
    ]
    if with_coeff:
        scratch.append(pltpu.VMEM((_ECR, _EC), jnp.float32))
    scratch += [
        pltpu.VMEM((_EC, _HH), jnp.float32),
        pltpu.VMEM_SHARED((_NP, _HH), jnp.float32),
        pltpu.SemaphoreType.DMA,
    ]

    @functools.partial(
        pl.kernel, mesh=mesh,
        out_type=jax.ShapeDtypeStruct((2, _NP, _HH), jnp.float32),
        scratch_types=scratch,
        compiler_params=pltpu.CompilerParams(use_tc_tiling_on_sc=False),
    )
    def k(vals0, vals1, src2, dst2, *rest):
        if with_coeff:
            cf2, zrs, out, src_v, dst_v, cf_v, rows, acc, sem = rest
        else:
            zrs, out, src_v, dst_v, rows, acc, sem = rest
        c = lax.axis_index("c")
        s = lax.axis_index("s")
        pltpu.sync_copy(src2.at[pl.ds(s * _ECR, _ECR)], src_v)
        pltpu.sync_copy(dst2.at[pl.ds(s * _ECR, _ECR)], dst_v)
        if with_coeff:
            pltpu.sync_copy(cf2.at[pl.ds(s * _ECR, _ECR)], cf_v)
        pltpu.sync_copy(zrs.at[pl.ds(s * _NPT, _NPT)],
                        acc.at[pl.ds(s * _NPT, _NPT)])
        plsc.subcore_barrier()

        def step(j, carry):
            @pl.when(c == 0)
            def _():
                pltpu.async_copy(vals0.at[src_v.at[j]], rows, sem).wait()

            @pl.when(c == 1)
            def _():
                pltpu.async_copy(vals1.at[src_v.at[j]], rows, sem).wait()

            if with_coeff:
                for r0 in range(0, _EC, 16):
                    cv = cf_v[j, pl.ds(r0, 16)]
                    for t in range(16):
                        ct = cv[t]
                        for h0 in range(0, _HH, 16):
                            sl = pl.ds(h0, 16)
                            rows[r0 + t, sl] = rows[r0 + t, sl] * ct
            pltpu.sync_copy(rows, acc.at[dst_v.at[j]], add=True)
            return carry

        lax.fori_loop(0, _ECR, step, 0)
        plsc.subcore_barrier()
        pltpu.sync_copy(acc.at[pl.ds(s * _NPT, _NPT)],
                        out.at[c, pl.ds(s * _NPT, _NPT)])

    return k


@functools.lru_cache(maxsize=None)
def _sc_agg(with_coeff):
    return _make_sc_agg(with_coeff)


def _max_tail(att, amax_ref):
    row = pl.program_id(0) * _BK + lax.broadcasted_iota(jnp.int32, (_BK, 1), 0)
    att = jnp.where(row < _N, att, -jnp.inf)
    m = jnp.max(att, axis=0, keepdims=True)

    @pl.when(pl.program_id(0) == 0)
    def _():
        amax_ref[...] = m

    @pl.when(pl.program_id(0) != 0)
    def _():
        amax_ref[...] = jnp.maximum(amax_ref[...], m)


def _pre_body(x_ref, w_ref, b_ref, dinv_ref, va_ref,
              xn_ref, xs_ref, att_ref, amax_ref):
    x = x_ref[...]
    y = jnp.dot(x, w_ref[...], preferred_element_type=jnp.float32) + b_ref[...]
    xn = jnp.maximum(y, 0.0)
    xn_ref[...] = xn
    xs_ref[...] = xn * dinv_ref[...]
    att = jnp.dot(xn, va_ref[...], preferred_element_type=jnp.float32)
    att_ref[...] = att
    _max_tail(att, amax_ref)


def _node_body(aggg_ref, xs_ref, agga_ref, dinv_ref,
               wg_ref, bg_ref, wa_ref, ba_ref, msk_ref, bsk_ref, va_ref,
               xn_ref, xsn_ref, att_ref, amax_ref):
    gg = aggg_ref[...]
    z = jnp.concatenate([gg[0], gg[1]], axis=1) + xs_ref[...]
    t1 = jnp.dot(z, wg_ref[...], preferred_element_type=jnp.float32)
    t1 = t1 * dinv_ref[...] + bg_ref[...]
    ga = agga_ref[...]
    a = jnp.concatenate([ga[0], ga[1]], axis=1)
    t2 = jnp.dot(a, wa_ref[...], preferred_element_type=jnp.float32) + ba_ref[...]
    o = t1 + t2
    xn = jnp.dot(o, msk_ref[...], preferred_element_type=jnp.float32) + bsk_ref[...]
    xn = jnp.maximum(xn, 0.0)
    xn_ref[...] = xn
    xsn_ref[...] = xn * dinv_ref[...]
    att = jnp.dot(xn, va_ref[...], preferred_element_type=jnp.float32)
    att_ref[...] = att
    _max_tail(att, amax_ref)


def _pool_body(x1_ref, b1_ref, x2_ref, b2_ref,
               wp1_ref, bp1_ref, wp2_ref, bp2_ref,
               p1_ref, c1_ref, p2_ref, c2_ref):
    ids = lax.broadcasted_iota(jnp.int32, (1, _NG), 1)

    def one(x_ref, b_ref, wp_ref, bp_ref, p_ref, c_ref):
        xp = jnp.dot(x_ref[...], wp_ref[...],
                     preferred_element_type=jnp.float32) + bp_ref[...]
        oh = (b_ref[...] == ids).astype(jnp.float32)
        ppart = lax.dot_general(oh, xp, (((0,), (0,)), ((), ())),
                                preferred_element_type=jnp.float32)
        cpart = jnp.sum(oh, axis=0, keepdims=True)

        @pl.when(pl.program_id(0) == 0)
        def _():
            p_ref[...] = ppart
            c_ref[...] = cpart

        @pl.when(pl.program_id(0) != 0)
        def _():
            p_ref[...] += ppart
            c_ref[...] += cpart

    one(x1_ref, b1_ref, wp1_ref, bp1_ref, p1_ref, c1_ref)
    one(x2_ref, b2_ref, wp2_ref, bp2_ref, p2_ref, c2_ref)


def _head_body(p1_ref, c1_ref, p2_ref, c2_ref,
               w1_ref, b1_ref, w2_ref, b2_ref, out_ref):
    c1 = jnp.maximum(c1_ref[...], 1.0)
    c2 = jnp.maximum(c2_ref[...], 1.0)
    r1 = p1_ref[...] * (1.0 / c1.T)
    r2 = p2_ref[...] * (1.0 / c2.T)
    r = jnp.concatenate([r1, r2], axis=1)
    h = jnp.dot(r, w1_ref[...], preferred_element_type=jnp.float32) + b1_ref[...]
    h = jnp.maximum(h, 0.0)
    out_ref[...] = jnp.dot(h, w2_ref[...],
                           preferred_element_type=jnp.float32) + b2_ref[...]


def _full(shape):
    return pl.BlockSpec(shape, lambda i: (0,) * len(shape))


def _rows(shape):
    return pl.BlockSpec(shape, lambda i: (i,) + (0,) * (len(shape) - 1))


def _pre_call(x, w, b, dinv, va):
    return pl.pallas_call(
        _pre_body,
        grid=(_NBLK,),
        in_specs=[_rows((_BK, _H)), _full((_H, _H)), _full((1, _H)),
                  _rows((_BK, 1)), _full((_H, 8))],
        out_specs=[_rows((_BK, _H)), _rows((_BK, _H)), _rows((_BK, 8)),
                   _full((1, 8))],
        out_shape=[jax.ShapeDtypeStruct((_NP, _H), jnp.float32),
                   jax.ShapeDtypeStruct((_NP, _H), jnp.float32),
                   jax.ShapeDtypeStruct((_NP, 8), jnp.float32),
                   jax.ShapeDtypeStruct((1, 8), jnp.float32)],
    )(x, w, b, dinv, va)


def _node_call(aggg, xs, agga, dinv, wg, bg, wa, ba, msk, bsk, va):
    return pl.pallas_call(
        _node_body,
        grid=(_NBLK,),
        in_specs=[
            pl.BlockSpec((2, _BK, _HH), lambda i: (0, i, 0)),
            _rows((_BK, _H)),
            pl.BlockSpec((2, _BK, _HH), lambda i: (0, i, 0)),
            _rows((_BK, 1)),
            _full((_H, _H)), _full((1, _H)),
            _full((_H, _H)), _full((1, _H)),
            _full((_H, _H)), _full((1, _H)),
            _full((_H, 8)),
        ],
        out_specs=[_rows((_BK, _H)), _rows((_BK, _H)), _rows((_BK, 8)),
                   _full((1, 8))],
        out_shape=[jax.ShapeDtypeStruct((_NP, _H), jnp.float32),
                   jax.ShapeDtypeStruct((_NP, _H), jnp.float32),
                   jax.ShapeDtypeStruct((_NP, 8), jnp.float32),
                   jax.ShapeDtypeStruct((1, 8), jnp.float32)],
    )(aggg, xs, agga, dinv, wg, bg, wa, ba, msk, bsk, va)


def _pool_call(x1, b1, x2, b2, wp1, bp1, wp2, bp2):
    return pl.pallas_call(
        _pool_body,
        grid=(_NBLK,),
        in_specs=[_rows((_BK, _H)), _rows((_BK, 1)),
                  _rows((_BK, _H)), _rows((_BK, 1)),
                  _full((_H, _H)), _full((1, _H)),
                  _full((_H, _H)), _full((1, _H))],
        out_specs=[_full((_NG, _H)), _full((1, _NG)),
                   _full((_NG, _H)), _full((1, _NG))],
        out_shape=[jax.ShapeDtypeStruct((_NG, _H), jnp.float32),
                   jax.ShapeDtypeStruct((1, _NG), jnp.float32),
                   jax.ShapeDtypeStruct((_NG, _H), jnp.float32),
                   jax.ShapeDtypeStruct((1, _NG), jnp.float32)],
    )(x1, b1, x2, b2, wp1, bp1, wp2, bp2)


def _head_call(p1, c1, p2, c2, w1, b1, w2, b2):
    return pl.pallas_call(
        _head_body,
        grid=(1,),
        in_specs=[_full((_NG, _H)), _full((1, _NG)),
                  _full((_NG, _H)), _full((1, _NG)),
                  _full((2 * _H, _H)), _full((1, _H)),
                  _full((_H, 32)), _full((1, 32))],
        out_specs=_full((_NG, 32)),
        out_shape=jax.ShapeDtypeStruct((_NG, 32), jnp.float32),
    )(p1, c1, p2, c2, w1, b1, w2, b2)


def _fold_bn(W, b, g, beta, m, v):
    inv = g * lax.rsqrt(v + 1e-5)
    return (W.T * inv[None, :], ((b - m) * inv + beta)[None, :])


def _lrelu(x):
    return jnp.maximum(x, 0.2 * x)


def _seg_sum_rows(vals, src, dst, coeff, n_dst):
    g = vals[src]
    if coeff is not None:
        g = g * coeff[:, None]
    return jnp.zeros((n_dst, vals.shape[1]), vals.dtype).at[dst].add(g)


def _edge_alpha(als, ald, src, dst, mshift, n_dst):
    e = _lrelu(als[src] + ald[dst])
    ex = jnp.exp(e - mshift)
    den = jnp.zeros((n_dst,), ex.dtype).at[dst].add(ex)
    return ex / jnp.maximum(den[dst], 1e-16)


def kernel(x_graph_1, x_graph_2, edge_index_g1_g1, edge_index_g2_g2,
           edge_index_g1_g2, edge_index_g2_g1, batch_graph_1, batch_graph_2,
           slice_dict, params):
    p = params
    L = 2

    wpre1, bpre1 = _fold_bn(p['pre_W_g1'], p['pre_b_g1'],
                            p['bn_pre_g1_gamma'], p['bn_pre_g1_beta'],
                            p['bn_pre_g1_mean'], p['bn_pre_g1_var'])
    wpre2, bpre2 = _fold_bn(p['pre_W_g2'], p['pre_b_g2'],
                            p['bn_pre_g2_gamma'], p['bn_pre_g2_beta'],
                            p['bn_pre_g2_mean'], p['bn_pre_g2_var'])
    wpost1, bpost1 = _fold_bn(p['post_W_g1'], p['post_b_g1'],
                              p['bn_post_g1_gamma'], p['bn_post_g1_beta'],
                              p['bn_post_g1_mean'], p['bn_post_g1_var'])
    wpost2, bpost2 = _fold_bn(p['post_W_g2'], p['post_b_g2'],
                              p['bn_post_g2_gamma'], p['bn_post_g2_beta'],
                              p['bn_post_g2_mean'], p['bn_post_g2_var'])

    eye = jnp.eye(_H, dtype=jnp.float32)
    zcol = jnp.zeros((_H, 1), jnp.float32)

    def att_w(l):
        v_as12 = p['gat12_Ws_%d' % l].T @ p['gat12_as_%d' % l]
        v_ad12 = p['gat12_Wd_%d' % l].T @ p['gat12_ad_%d' % l]
        v_as21 = p['gat21_Ws_%d' % l].T @ p['gat21_as_%d' % l]
        v_ad21 = p['gat21_Wd_%d' % l].T @ p['gat21_ad_%d' % l]
        va1 = jnp.concatenate([v_as12[:, None], v_ad21[:, None]] + [zcol] * 6,
                              axis=1)
        va2 = jnp.concatenate([v_as21[:, None], v_ad12[:, None]] + [zcol] * 6,
                              axis=1)
        return va1, va2

    va_zero = jnp.zeros((_H, 8), jnp.float32)

    src11, dst11 = edge_index_g1_g1[0], edge_index_g1_g1[1]
    src22, dst22 = edge_index_g2_g2[0], edge_index_g2_g2[1]
    src12, dst12 = edge_index_g1_g2[0], edge_index_g1_g2[1]
    src21, dst21 = edge_index_g2_g1[0], edge_index_g2_g1[1]

    deg1 = jnp.zeros((_NP,), jnp.float32).at[dst11].add(1.0) + 1.0
    deg2 = jnp.zeros((_NP,), jnp.float32).at[dst22].add(1.0) + 1.0
    dinv1 = lax.rsqrt(deg1)[:, None]
    dinv2 = lax.rsqrt(deg2)[:, None]

    npad = _NP - _N
    x1in = jnp.pad(x_graph_1, ((0, npad), (0, 0)))
    x2in = jnp.pad(x_graph_2, ((0, npad), (0, 0)))
    va1, va2 = att_w(0)
    x1, xs1, att1, am1 = _pre_call(x1in, wpre1, bpre1, dinv1, va1)
    x2, xs2, att2, am2 = _pre_call(x2in, wpre2, bpre2, dinv2, va2)

    zrows = jnp.zeros((_NP, _HH), jnp.float32)
    epad = _E2D * _EC - _E
    spad = jnp.zeros((epad,), jnp.int32)
    dpad = jnp.full((epad,), _N, jnp.int32)
    cpad = jnp.zeros((epad,), jnp.float32)

    def e2d(v, pad):
        return jnp.concatenate([v, pad]).reshape(_E2D, _EC)

    s11_2d, d11_2d = e2d(src11, spad), e2d(dst11, dpad)
    s22_2d, d22_2d = e2d(src22, spad), e2d(dst22, dpad)
    s12_2d, d12_2d = e2d(src12, spad), e2d(dst12, dpad)
    s21_2d, d21_2d = e2d(src21, spad), e2d(dst21, dpad)

    for l in range(L):
        aggg1 = _sc_agg(False)(xs1[:, :_HH], xs1[:, _HH:], s11_2d, d11_2d,
                               zrows)
        aggg2 = _sc_agg(False)(xs2[:, :_HH], xs2[:, _HH:], s22_2d, d22_2d,
                               zrows)

        m12 = _lrelu(am1[0, 0] + am2[0, 1])
        m21 = _lrelu(am2[0, 0] + am1[0, 1])
        alpha12 = _edge_alpha(att1[:, 0], att2[:, 1], src12, dst12, m12, _N)
        alpha21 = _edge_alpha(att2[:, 0], att1[:, 1], src21, dst21, m21, _N)
        agga1 = _sc_agg(True)(x2[:, :_HH], x2[:, _HH:], s21_2d, d21_2d,
                              e2d(alpha21, cpad), zrows)
        agga2 = _sc_agg(True)(x1[:, :_HH], x1[:, _HH:], s12_2d, d12_2d,
                              e2d(alpha12, cpad), zrows)

        van1, van2 = att_w(l + 1) if l + 1 < L else (va_zero, va_zero)
        x1, xs1, att1, am1 = _node_call(
            aggg1, xs1, agga1, dinv1,
            p['gcn11_W_%d' % l].T, p['gcn11_b_%d' % l][None, :],
            p['gat21_Ws_%d' % l].T, p['gat21_b_%d' % l][None, :],
            eye + p['skip_W_g1_%d' % l].T, p['skip_b_g1_%d' % l][None, :],
            van1)
        x2, xs2, att2, am2 = _node_call(
            aggg2, xs2, agga2, dinv2,
            p['gcn22_W_%d' % l].T, p['gcn22_b_%d' % l][None, :],
            p['gat12_Ws_%d' % l].T, p['gat12_b_%d' % l][None, :],
            eye + p['skip_W_g2_%d' % l].T, p['skip_b_g2_%d' % l][None, :],
            van2)

    b1p = jnp.pad(batch_graph_1, (0, npad), constant_values=_NG)[:, None]
    b2p = jnp.pad(batch_graph_2, (0, npad), constant_values=_NG)[:, None]
    p1, c1, p2, c2 = _pool_call(x1, b1p, x2, b2p,
                                wpost1, bpost1, wpost2, bpost2)
    return _head_call(p1, c1, p2, c2,
                      p['lin1_W'].T, p['lin1_b'][None, :],
                      p['lin2_W'].T, p['lin2_b'][None, :])


# speedup vs baseline: 10.0858x; 4.6388x over previous
import functools

import jax
import jax.numpy as jnp
from jax import lax
from jax.experimental import pallas as pl
from jax.experimental.pallas import tpu as pltpu
from jax.experimental.pallas import tpu_sc as plsc

_N = 10000
_NP = 10240
_H = 128
_BK = 1024
_NG = 16
_NBLK = _NP // _BK

_E = 320000
_EC = 80
_E2D = 4096
_ECR = _E2D // 16
_NPT = _NP // 16
_HH = _H // 2


def _make_sc_agg(with_coeff):
    mesh = plsc.VectorSubcoreMesh(core_axis_name="c", subcore_axis_name="s")
    scratch = [
        pltpu.VMEM((_ECR, _EC), jnp.int32),
        pltpu.VMEM((_ECR, _EC), jnp.int32),
    ]
    if with_coeff:
        scratch.append(pltpu.VMEM((_ECR, _EC), jnp.float32))
    scratch += [
        pltpu.VMEM((_EC, _HH), jnp.float32),
        pltpu.VMEM_SHARED((_NP, _HH), jnp.float32),
        pltpu.SemaphoreType.DMA,
    ]

    @functools.partial(
        pl.kernel, mesh=mesh,
        out_type=jax.ShapeDtypeStruct((2, _NP, _HH), jnp.float32),
        scratch_types=scratch,
        compiler_params=pltpu.CompilerParams(use_tc_tiling_on_sc=False),
    )
    def k(vals0, vals1, src2, dst2, *rest):
        if with_coeff:
            cf2, zrs, out, src_v, dst_v, cf_v, rows, acc, sem = rest
        else:
            zrs, out, src_v, dst_v, rows, acc, sem = rest
        c = lax.axis_index("c")
        s = lax.axis_index("s")
        pltpu.sync_copy(src2.at[pl.ds(s * _ECR, _ECR)], src_v)
        pltpu.sync_copy(dst2.at[pl.ds(s * _ECR, _ECR)], dst_v)
        if with_coeff:
            pltpu.sync_copy(cf2.at[pl.ds(s * _ECR, _ECR)], cf_v)
        pltpu.sync_copy(zrs.at[pl.ds(s * _NPT, _NPT)],
                        acc.at[pl.ds(s * _NPT, _NPT)])
        plsc.subcore_barrier()

        def step(j, carry):
            @pl.when(c == 0)
            def _():
                pltpu.async_copy(vals0.at[src_v.at[j]], rows, sem).wait()

            @pl.when(c == 1)
            def _():
                pltpu.async_copy(vals1.at[src_v.at[j]], rows, sem).wait()

            if with_coeff:
                for r0 in range(0, _EC, 16):
                    cv = cf_v[j, pl.ds(r0, 16)]
                    for t in range(16):
                        ct = cv[t]
                        for h0 in range(0, _HH, 16):
                            sl = pl.ds(h0, 16)
                            rows[r0 + t, sl] = rows[r0 + t, sl] * ct
            pltpu.sync_copy(rows, acc.at[dst_v.at[j]], add=True)
            return carry

        lax.fori_loop(0, _ECR, step, 0)
        plsc.subcore_barrier()
        pltpu.sync_copy(acc.at[pl.ds(s * _NPT, _NPT)],
                        out.at[c, pl.ds(s * _NPT, _NPT)])

    return k


@functools.lru_cache(maxsize=None)
def _sc_agg(with_coeff):
    return _make_sc_agg(with_coeff)


def _make_sc_alpha():
    mesh = plsc.VectorSubcoreMesh(core_axis_name="c", subcore_axis_name="s")
    scratch = [
        pltpu.VMEM((_ECR, _EC), jnp.int32),
        pltpu.VMEM((_ECR, _EC), jnp.int32),
        pltpu.VMEM((_ECR, _EC), jnp.float32),
        pltpu.VMEM((_EC,), jnp.float32),
        pltpu.VMEM((_EC,), jnp.float32),
        pltpu.VMEM((640,), jnp.float32),
        pltpu.VMEM((16,), jnp.float32),
        pltpu.VMEM_SHARED((_NP,), jnp.float32),
        pltpu.SemaphoreType.DMA,
        pltpu.SemaphoreType.DMA,
    ]

    @functools.partial(
        pl.kernel, mesh=mesh,
        out_type=jax.ShapeDtypeStruct((_E2D, _EC), jnp.float32),
        scratch_types=scratch,
    )
    def k(als, ald, src2, dst2, mb, out,
          src_v, dst_v, ex_v, ga_v, gb_v, zb_v, mb_v, denf, sem, sem2):
        c = lax.axis_index("c")
        s = lax.axis_index("s")
        pltpu.sync_copy(src2.at[pl.ds(s * _ECR, _ECR)], src_v)
        pltpu.sync_copy(dst2.at[pl.ds(s * _ECR, _ECR)], dst_v)
        pltpu.sync_copy(mb, mb_v)
        z16 = jnp.zeros((16,), jnp.float32)

        def zstep(j, carry):
            zb_v[pl.ds(j * 16, 16)] = z16
            return carry

        lax.fori_loop(0, 40, zstep, 0)
        pltpu.sync_copy(zb_v, denf.at[pl.ds(s * 640, 640)])
        mbv = mb_v[...]
        plsc.subcore_barrier()

        def step(j, carry):
            h1 = pltpu.async_copy(als.at[src_v.at[j]], ga_v, sem)
            h2 = pltpu.async_copy(ald.at[dst_v.at[j]], gb_v, sem2)
            h1.wait()
            h2.wait()
            for u in range(5):
                sl = pl.ds(16 * u, 16)
                t = ga_v[sl] + gb_v[sl]
                e = jnp.maximum(t, 0.2 * t)
                ex_v[j, sl] = jnp.exp(e - mbv)
            pltpu.sync_copy(ex_v.at[j], denf.at[dst_v.at[j]], add=True)
            return carry

        lax.fori_loop(0, _ECR, step, 0)
        plsc.subcore_barrier()
        base = c * (_ECR // 2)

        def step2(i, carry):
            j = base + i
            pltpu.async_copy(denf.at[dst_v.at[j]], gb_v, sem2).wait()
            for u in range(5):
                sl = pl.ds(16 * u, 16)
                den = gb_v[sl]
                ex_v[j, sl] = ex_v[j, sl] / jnp.maximum(den, 1e-16)
            return carry

        lax.fori_loop(0, _ECR // 2, step2, 0)
        pltpu.sync_copy(ex_v.at[pl.ds(base, _ECR // 2)],
                        out.at[pl.ds(s * _ECR + base, _ECR // 2)])

    return k


def _make_sc_count():
    mesh = plsc.VectorSubcoreMesh(core_axis_name="c", subcore_axis_name="s")
    scratch = [
        pltpu.VMEM((_ECR, _EC), jnp.int32),
        pltpu.VMEM((_EC,), jnp.float32),
        pltpu.VMEM((640,), jnp.float32),
        pltpu.VMEM_SHARED((_NP,), jnp.float32),
        pltpu.SemaphoreType.DMA,
        pltpu.SemaphoreType.DMA,
    ]

    @functools.partial(
        pl.kernel, mesh=mesh,
        out_type=jax.ShapeDtypeStruct((2, _NP), jnp.float32),
        scratch_types=scratch,
    )
    def k(dst2, out, dst_v, one_v, zb_v, denf, sem, sem2):
        c = lax.axis_index("c")
        s = lax.axis_index("s")
        pltpu.sync_copy(dst2.at[pl.ds(s * _ECR, _ECR)], dst_v)
        z16 = jnp.zeros((16,), jnp.float32)
        o16 = jnp.ones((16,), jnp.float32)
        for u in range(5):
            one_v[pl.ds(16 * u, 16)] = o16

        def zstep(j, carry):
            zb_v[pl.ds(j * 16, 16)] = z16
            return carry

        lax.fori_loop(0, 40, zstep, 0)
        pltpu.sync_copy(zb_v, denf.at[pl.ds(s * 640, 640)])
        plsc.subcore_barrier()

        def step(j, carry):
            pltpu.sync_copy(one_v, denf.at[dst_v.at[j]], add=True)
            return carry

        lax.fori_loop(0, _ECR, step, 0)
        plsc.subcore_barrier()
        pltpu.sync_copy(denf.at[pl.ds(s * 640, 640)],
                        out.at[c, pl.ds(s * 640, 640)])

    return k


@functools.lru_cache(maxsize=None)
def _sc_alpha_k():
    return _make_sc_alpha()


@functools.lru_cache(maxsize=None)
def _sc_count_k():
    return _make_sc_count()


def _max_tail(att, amax_ref):
    row = pl.program_id(0) * _BK + lax.broadcasted_iota(jnp.int32, (_BK, 1), 0)
    att = jnp.where(row < _N, att, -jnp.inf)
    m = jnp.max(att, axis=0, keepdims=True)

    @pl.when(pl.program_id(0) == 0)
    def _():
        amax_ref[...] = m

    @pl.when(pl.program_id(0) != 0)
    def _():
        amax_ref[...] = jnp.maximum(amax_ref[...], m)


def _pre_body(x_ref, w_ref, b_ref, dinv_ref, va_ref,
              xn_ref, xs_ref, att_ref, amax_ref):
    x = x_ref[...]
    y = jnp.dot(x, w_ref[...], preferred_element_type=jnp.float32) + b_ref[...]
    xn = jnp.maximum(y, 0.0)
    xn_ref[...] = xn
    xs_ref[...] = xn * dinv_ref[...]
    att = jnp.dot(xn, va_ref[...], preferred_element_type=jnp.float32)
    att_ref[...] = att
    _max_tail(att, amax_ref)


def _node_body(aggg_ref, xs_ref, agga_ref, dinv_ref,
               wg_ref, bg_ref, wa_ref, ba_ref, msk_ref, bsk_ref, va_ref,
               xn_ref, xsn_ref, att_ref, amax_ref):
    gg = aggg_ref[...]
    z = jnp.concatenate([gg[0], gg[1]], axis=1) + xs_ref[...]
    t1 = jnp.dot(z, wg_ref[...], preferred_element_type=jnp.float32)
    t1 = t1 * dinv_ref[...] + bg_ref[...]
    ga = agga_ref[...]
    a = jnp.concatenate([ga[0], ga[1]], axis=1)
    t2 = jnp.dot(a, wa_ref[...], preferred_element_type=jnp.float32) + ba_ref[...]
    o = t1 + t2
    xn = jnp.dot(o, msk_ref[...], preferred_element_type=jnp.float32) + bsk_ref[...]
    xn = jnp.maximum(xn, 0.0)
    xn_ref[...] = xn
    xsn_ref[...] = xn * dinv_ref[...]
    att = jnp.dot(xn, va_ref[...], preferred_element_type=jnp.float32)
    att_ref[...] = att
    _max_tail(att, amax_ref)


def _pool_body(x1_ref, b1_ref, x2_ref, b2_ref,
               wp1_ref, bp1_ref, wp2_ref, bp2_ref,
               p1_ref, c1_ref, p2_ref, c2_ref):
    ids = lax.broadcasted_iota(jnp.int32, (1, _NG), 1)

    def one(x_ref, b_ref, wp_ref, bp_ref, p_ref, c_ref):
        xp = jnp.dot(x_ref[...], wp_ref[...],
                     preferred_element_type=jnp.float32) + bp_ref[...]
        oh = (b_ref[...] == ids).astype(jnp.float32)
        ppart = lax.dot_general(oh, xp, (((0,), (0,)), ((), ())),
                                preferred_element_type=jnp.float32)
        cpart = jnp.sum(oh, axis=0, keepdims=True)

        @pl.when(pl.program_id(0) == 0)
        def _():
            p_ref[...] = ppart
            c_ref[...] = cpart

        @pl.when(pl.program_id(0) != 0)
        def _():
            p_ref[...] += ppart
            c_ref[...] += cpart

    one(x1_ref, b1_ref, wp1_ref, bp1_ref, p1_ref, c1_ref)
    one(x2_ref, b2_ref, wp2_ref, bp2_ref, p2_ref, c2_ref)


def _head_body(p1_ref, c1_ref, p2_ref, c2_ref,
               w1_ref, b1_ref, w2_ref, b2_ref, out_ref):
    c1 = jnp.maximum(c1_ref[...], 1.0)
    c2 = jnp.maximum(c2_ref[...], 1.0)
    r1 = p1_ref[...] * (1.0 / c1.T)
    r2 = p2_ref[...] * (1.0 / c2.T)
    r = jnp.concatenate([r1, r2], axis=1)
    h = jnp.dot(r, w1_ref[...], preferred_element_type=jnp.float32) + b1_ref[...]
    h = jnp.maximum(h, 0.0)
    out_ref[...] = jnp.dot(h, w2_ref[...],
                           preferred_element_type=jnp.float32) + b2_ref[...]


def _full(shape):
    return pl.BlockSpec(shape, lambda i: (0,) * len(shape))


def _rows(shape):
    return pl.BlockSpec(shape, lambda i: (i,) + (0,) * (len(shape) - 1))


def _pre_call(x, w, b, dinv, va):
    return pl.pallas_call(
        _pre_body,
        grid=(_NBLK,),
        in_specs=[_rows((_BK, _H)), _full((_H, _H)), _full((1, _H)),
                  _rows((_BK, 1)), _full((_H, 8))],
        out_specs=[_rows((_BK, _H)), _rows((_BK, _H)), _rows((_BK, 8)),
                   _full((1, 8))],
        out_shape=[jax.ShapeDtypeStruct((_NP, _H), jnp.float32),
                   jax.ShapeDtypeStruct((_NP, _H), jnp.float32),
                   jax.ShapeDtypeStruct((_NP, 8), jnp.float32),
                   jax.ShapeDtypeStruct((1, 8), jnp.float32)],
    )(x, w, b, dinv, va)


def _node_call(aggg, xs, agga, dinv, wg, bg, wa, ba, msk, bsk, va):
    return pl.pallas_call(
        _node_body,
        grid=(_NBLK,),
        in_specs=[
            pl.BlockSpec((2, _BK, _HH), lambda i: (0, i, 0)),
            _rows((_BK, _H)),
            pl.BlockSpec((2, _BK, _HH), lambda i: (0, i, 0)),
            _rows((_BK, 1)),
            _full((_H, _H)), _full((1, _H)),
            _full((_H, _H)), _full((1, _H)),
            _full((_H, _H)), _full((1, _H)),
            _full((_H, 8)),
        ],
        out_specs=[_rows((_BK, _H)), _rows((_BK, _H)), _rows((_BK, 8)),
                   _full((1, 8))],
        out_shape=[jax.ShapeDtypeStruct((_NP, _H), jnp.float32),
                   jax.ShapeDtypeStruct((_NP, _H), jnp.float32),
                   jax.ShapeDtypeStruct((_NP, 8), jnp.float32),
                   jax.ShapeDtypeStruct((1, 8), jnp.float32)],
    )(aggg, xs, agga, dinv, wg, bg, wa, ba, msk, bsk, va)


def _pool_call(x1, b1, x2, b2, wp1, bp1, wp2, bp2):
    return pl.pallas_call(
        _pool_body,
        grid=(_NBLK,),
        in_specs=[_rows((_BK, _H)), _rows((_BK, 1)),
                  _rows((_BK, _H)), _rows((_BK, 1)),
                  _full((_H, _H)), _full((1, _H)),
                  _full((_H, _H)), _full((1, _H))],
        out_specs=[_full((_NG, _H)), _full((1, _NG)),
                   _full((_NG, _H)), _full((1, _NG))],
        out_shape=[jax.ShapeDtypeStruct((_NG, _H), jnp.float32),
                   jax.ShapeDtypeStruct((1, _NG), jnp.float32),
                   jax.ShapeDtypeStruct((_NG, _H), jnp.float32),
                   jax.ShapeDtypeStruct((1, _NG), jnp.float32)],
    )(x1, b1, x2, b2, wp1, bp1, wp2, bp2)


def _head_call(p1, c1, p2, c2, w1, b1, w2, b2):
    return pl.pallas_call(
        _head_body,
        grid=(1,),
        in_specs=[_full((_NG, _H)), _full((1, _NG)),
                  _full((_NG, _H)), _full((1, _NG)),
                  _full((2 * _H, _H)), _full((1, _H)),
                  _full((_H, 32)), _full((1, 32))],
        out_specs=_full((_NG, 32)),
        out_shape=jax.ShapeDtypeStruct((_NG, 32), jnp.float32),
    )(p1, c1, p2, c2, w1, b1, w2, b2)


def _fold_bn(W, b, g, beta, m, v):
    inv = g * lax.rsqrt(v + 1e-5)
    return (W.T * inv[None, :], ((b - m) * inv + beta)[None, :])


def _lrelu(x):
    return jnp.maximum(x, 0.2 * x)


def _seg_sum_rows(vals, src, dst, coeff, n_dst):
    g = vals[src]
    if coeff is not None:
        g = g * coeff[:, None]
    return jnp.zeros((n_dst, vals.shape[1]), vals.dtype).at[dst].add(g)


def _edge_alpha(als, ald, src, dst, mshift, n_dst):
    e = _lrelu(als[src] + ald[dst])
    ex = jnp.exp(e - mshift)
    den = jnp.zeros((n_dst,), ex.dtype).at[dst].add(ex)
    return ex / jnp.maximum(den[dst], 1e-16)


def kernel(x_graph_1, x_graph_2, edge_index_g1_g1, edge_index_g2_g2,
           edge_index_g1_g2, edge_index_g2_g1, batch_graph_1, batch_graph_2,
           slice_dict, params):
    p = params
    L = 2

    wpre1, bpre1 = _fold_bn(p['pre_W_g1'], p['pre_b_g1'],
                            p['bn_pre_g1_gamma'], p['bn_pre_g1_beta'],
                            p['bn_pre_g1_mean'], p['bn_pre_g1_var'])
    wpre2, bpre2 = _fold_bn(p['pre_W_g2'], p['pre_b_g2'],
                            p['bn_pre_g2_gamma'], p['bn_pre_g2_beta'],
                            p['bn_pre_g2_mean'], p['bn_pre_g2_var'])
    wpost1, bpost1 = _fold_bn(p['post_W_g1'], p['post_b_g1'],
                              p['bn_post_g1_gamma'], p['bn_post_g1_beta'],
                              p['bn_post_g1_mean'], p['bn_post_g1_var'])
    wpost2, bpost2 = _fold_bn(p['post_W_g2'], p['post_b_g2'],
                              p['bn_post_g2_gamma'], p['bn_post_g2_beta'],
                              p['bn_post_g2_mean'], p['bn_post_g2_var'])

    eye = jnp.eye(_H, dtype=jnp.float32)
    zcol = jnp.zeros((_H, 1), jnp.float32)

    def att_w(l):
        v_as12 = p['gat12_Ws_%d' % l].T @ p['gat12_as_%d' % l]
        v_ad12 = p['gat12_Wd_%d' % l].T @ p['gat12_ad_%d' % l]
        v_as21 = p['gat21_Ws_%d' % l].T @ p['gat21_as_%d' % l]
        v_ad21 = p['gat21_Wd_%d' % l].T @ p['gat21_ad_%d' % l]
        va1 = jnp.concatenate([v_as12[:, None], v_ad21[:, None]] + [zcol] * 6,
                              axis=1)
        va2 = jnp.concatenate([v_as21[:, None], v_ad12[:, None]] + [zcol] * 6,
                              axis=1)
        return va1, va2

    va_zero = jnp.zeros((_H, 8), jnp.float32)

    src11, dst11 = edge_index_g1_g1[0], edge_index_g1_g1[1]
    src22, dst22 = edge_index_g2_g2[0], edge_index_g2_g2[1]
    src12, dst12 = edge_index_g1_g2[0], edge_index_g1_g2[1]
    src21, dst21 = edge_index_g2_g1[0], edge_index_g2_g1[1]

    zrows = jnp.zeros((_NP, _HH), jnp.float32)
    epad = _E2D * _EC - _E
    spad = jnp.zeros((epad,), jnp.int32)
    dpad = jnp.full((epad,), _N, jnp.int32)

    def e2d(v, pad):
        return jnp.concatenate([v, pad]).reshape(_E2D, _EC)

    s11_2d, d11_2d = e2d(src11, spad), e2d(dst11, dpad)
    s22_2d, d22_2d = e2d(src22, spad), e2d(dst22, dpad)
    s12_2d, d12_2d = e2d(src12, spad), e2d(dst12, dpad)
    s21_2d, d21_2d = e2d(src21, spad), e2d(dst21, dpad)

    deg1 = _sc_count_k()(d11_2d)[0] + 1.0
    deg2 = _sc_count_k()(d22_2d)[0] + 1.0
    dinv1 = lax.rsqrt(deg1)[:, None]
    dinv2 = lax.rsqrt(deg2)[:, None]

    npad = _NP - _N
    x1in = jnp.pad(x_graph_1, ((0, npad), (0, 0)))
    x2in = jnp.pad(x_graph_2, ((0, npad), (0, 0)))
    va1, va2 = att_w(0)
    x1, xs1, att1, am1 = _pre_call(x1in, wpre1, bpre1, dinv1, va1)
    x2, xs2, att2, am2 = _pre_call(x2in, wpre2, bpre2, dinv2, va2)

    for l in range(L):
        aggg1 = _sc_agg(False)(xs1[:, :_HH], xs1[:, _HH:], s11_2d, d11_2d,
                               zrows)
        aggg2 = _sc_agg(False)(xs2[:, :_HH], xs2[:, _HH:], s22_2d, d22_2d,
                               zrows)

        m12 = _lrelu(am1[0, 0] + am2[0, 1])
        m21 = _lrelu(am2[0, 0] + am1[0, 1])
        al12 = _sc_alpha_k()(att1[:, 0], att2[:, 1], s12_2d, d12_2d,
                             jnp.full((16,), m12, jnp.float32))
        al21 = _sc_alpha_k()(att2[:, 0], att1[:, 1], s21_2d, d21_2d,
                             jnp.full((16,), m21, jnp.float32))
        agga1 = _sc_agg(True)(x2[:, :_HH], x2[:, _HH:], s21_2d, d21_2d,
                              al21, zrows)
        agga2 = _sc_agg(True)(x1[:, :_HH], x1[:, _HH:], s12_2d, d12_2d,
                              al12, zrows)

        van1, van2 = att_w(l + 1) if l + 1 < L else (va_zero, va_zero)
        x1, xs1, att1, am1 = _node_call(
            aggg1, xs1, agga1, dinv1,
            p['gcn11_W_%d' % l].T, p['gcn11_b_%d' % l][None, :],
            p['gat21_Ws_%d' % l].T, p['gat21_b_%d' % l][None, :],
            eye + p['skip_W_g1_%d' % l].T, p['skip_b_g1_%d' % l][None, :],
            van1)
        x2, xs2, att2, am2 = _node_call(
            aggg2, xs2, agga2, dinv2,
            p['gcn22_W_%d' % l].T, p['gcn22_b_%d' % l][None, :],
            p['gat12_Ws_%d' % l].T, p['gat12_b_%d' % l][None, :],
            eye + p['skip_W_g2_%d' % l].T, p['skip_b_g2_%d' % l][None, :],
            van2)

    b1p = jnp.pad(batch_graph_1, (0, npad), constant_values=_NG)[:, None]
    b2p = jnp.pad(batch_graph_2, (0, npad), constant_values=_NG)[:, None]
    p1, c1, p2, c2 = _pool_call(x1, b1p, x2, b2p,
                                wpost1, bpost1, wpost2, bpost2)
    return _head_call(p1, c1, p2, c2,
                      p['lin1_W'].T, p['lin1_b'][None, :],
                      p['lin2_W'].T, p['lin2_b'][None, :])


# speedup vs baseline: 13.4716x; 1.3357x over previous
import functools

import jax
import jax.numpy as jnp
from jax import lax
from jax.experimental import pallas as pl
from jax.experimental.pallas import tpu as pltpu
from jax.experimental.pallas import tpu_sc as plsc

_N = 10000
_NP = 10240
_H = 128
_BK = 1024
_NG = 16
_NBLK = _NP // _BK

_E = 320000
_EC = 80
_E2D = 4096
_ECR = _E2D // 16
_NPT = _NP // 16
_HH = _H // 2


def _make_sc_agg(with_coeff):
    mesh = plsc.VectorSubcoreMesh(core_axis_name="c", subcore_axis_name="s")
    scratch = [
        pltpu.VMEM((_ECR, _EC), jnp.int32),
        pltpu.VMEM((_ECR, _EC), jnp.int32),
    ]
    if with_coeff:
        scratch.append(pltpu.VMEM((_ECR, _EC), jnp.float32))
    scratch += [
        pltpu.VMEM((2, _EC, _HH), jnp.float32),
        pltpu.VMEM((2, _EC, _HH), jnp.float32),
        pltpu.VMEM_SHARED((_NP, _HH), jnp.float32),
        pltpu.SemaphoreType.DMA,
        pltpu.SemaphoreType.DMA,
        pltpu.SemaphoreType.DMA,
        pltpu.SemaphoreType.DMA,
    ]

    @functools.partial(
        pl.kernel, mesh=mesh,
        out_type=jax.ShapeDtypeStruct((2, _NP, _HH), jnp.float32),
        scratch_types=scratch,
        compiler_params=pltpu.CompilerParams(use_tc_tiling_on_sc=False),
    )
    def k(vals0, vals1, src2, dst2, *rest):
        if with_coeff:
            cf2, zrs, out, src_v, dst_v, cf_v, gbuf, sbuf, acc, \
                gs0, gs1, ss0, ss1 = rest
        else:
            zrs, out, src_v, dst_v, gbuf, sbuf, acc, gs0, gs1, ss0, ss1 = rest
        c = lax.axis_index("c")
        s = lax.axis_index("s")
        gsem = (gs0, gs1)
        ssem = (ss0, ss1)
        pltpu.sync_copy(src2.at[pl.ds(s * _ECR, _ECR)], src_v)
        pltpu.sync_copy(dst2.at[pl.ds(s * _ECR, _ECR)], dst_v)
        if with_coeff:
            pltpu.sync_copy(cf2.at[pl.ds(s * _ECR, _ECR)], cf_v)
        pltpu.sync_copy(zrs.at[pl.ds(s * _NPT, _NPT)],
                        acc.at[pl.ds(s * _NPT, _NPT)])
        plsc.subcore_barrier()

        def fire_gather(j, b):
            @pl.when(c == 0)
            def _():
                pltpu.async_copy(vals0.at[src_v.at[j]], gbuf.at[b], gsem[b])

            @pl.when(c == 1)
            def _():
                pltpu.async_copy(vals1.at[src_v.at[j]], gbuf.at[b], gsem[b])

        def drain(ref_b, sem):
            pltpu.make_async_copy(vals0.at[pl.ds(0, _EC)], ref_b, sem).wait()

        fire_gather(0, 0)
        fire_gather(1, 1)
        nhalf = _ECR // 2

        def step(i, carry):
            for b in range(2):
                j = 2 * i + b
                drain(gbuf.at[b], gsem[b])

                @pl.when(i > 0)
                def _():
                    drain(sbuf.at[b], ssem[b])

                if with_coeff:
                    for r0 in range(0, _EC, 16):
                        cv = cf_v[j, pl.ds(r0, 16)]
                        for r in range(16):
                            cr = cv[r]
                            for h0 in range(0, _HH, 16):
                                sl = pl.ds(h0, 16)
                                sbuf[b, r0 + r, sl] = gbuf[b, r0 + r, sl] * cr
                else:
                    for r in range(_EC):
                        for h0 in range(0, _HH, 16):
                            sl = pl.ds(h0, 16)
                            sbuf[b, r, sl] = gbuf[b, r, sl]
                pltpu.async_copy(sbuf.at[b], acc.at[dst_v.at[j]], ssem[b],
                                 add=True)

                @pl.when(i < nhalf - 1)
                def _():
                    fire_gather(j + 2, b)
            return carry

        lax.fori_loop(0, nhalf, step, 0)
        drain(sbuf.at[0], ssem[0])
        drain(sbuf.at[1], ssem[1])
        plsc.subcore_barrier()
        pltpu.sync_copy(acc.at[pl.ds(s * _NPT, _NPT)],
                        out.at[c, pl.ds(s * _NPT, _NPT)])

    return k


@functools.lru_cache(maxsize=None)
def _sc_agg(with_coeff):
    return _make_sc_agg(with_coeff)


def _make_sc_alpha():
    mesh = plsc.VectorSubcoreMesh(core_axis_name="c", subcore_axis_name="s")
    scratch = [
        pltpu.VMEM((_ECR, _EC), jnp.int32),
        pltpu.VMEM((_ECR, _EC), jnp.int32),
        pltpu.VMEM((_ECR, _EC), jnp.float32),
        pltpu.VMEM((_EC,), jnp.float32),
        pltpu.VMEM((_EC,), jnp.float32),
        pltpu.VMEM((640,), jnp.float32),
        pltpu.VMEM((16,), jnp.float32),
        pltpu.VMEM_SHARED((_NP,), jnp.float32),
        pltpu.SemaphoreType.DMA,
        pltpu.SemaphoreType.DMA,
    ]

    @functools.partial(
        pl.kernel, mesh=mesh,
        out_type=jax.ShapeDtypeStruct((_E2D, _EC), jnp.float32),
        scratch_types=scratch,
    )
    def k(als, ald, src2, dst2, mb, out,
          src_v, dst_v, ex_v, ga_v, gb_v, zb_v, mb_v, denf, sem, sem2):
        c = lax.axis_index("c")
        s = lax.axis_index("s")
        pltpu.sync_copy(src2.at[pl.ds(s * _ECR, _ECR)], src_v)
        pltpu.sync_copy(dst2.at[pl.ds(s * _ECR, _ECR)], dst_v)
        pltpu.sync_copy(mb, mb_v)
        z16 = jnp.zeros((16,), jnp.float32)

        def zstep(j, carry):
            zb_v[pl.ds(j * 16, 16)] = z16
            return carry

        lax.fori_loop(0, 40, zstep, 0)
        pltpu.sync_copy(zb_v, denf.at[pl.ds(s * 640, 640)])
        mbv = mb_v[...]
        plsc.subcore_barrier()

        def step(j, carry):
            h1 = pltpu.async_copy(als.at[src_v.at[j]], ga_v, sem)
            h2 = pltpu.async_copy(ald.at[dst_v.at[j]], gb_v, sem2)
            h1.wait()
            h2.wait()
            for u in range(5):
                sl = pl.ds(16 * u, 16)
                t = ga_v[sl] + gb_v[sl]
                e = jnp.maximum(t, 0.2 * t)
                ex_v[j, sl] = jnp.exp(e - mbv)
            pltpu.sync_copy(ex_v.at[j], denf.at[dst_v.at[j]], add=True)
            return carry

        lax.fori_loop(0, _ECR, step, 0)
        plsc.subcore_barrier()
        base = c * (_ECR // 2)

        def step2(i, carry):
            j = base + i
            pltpu.async_copy(denf.at[dst_v.at[j]], gb_v, sem2).wait()
            for u in range(5):
                sl = pl.ds(16 * u, 16)
                den = gb_v[sl]
                ex_v[j, sl] = ex_v[j, sl] / jnp.maximum(den, 1e-16)
            return carry

        lax.fori_loop(0, _ECR // 2, step2, 0)
        pltpu.sync_copy(ex_v.at[pl.ds(base, _ECR // 2)],
                        out.at[pl.ds(s * _ECR + base, _ECR // 2)])

    return k


def _make_sc_count():
    mesh = plsc.VectorSubcoreMesh(core_axis_name="c", subcore_axis_name="s")
    scratch = [
        pltpu.VMEM((_ECR, _EC), jnp.int32),
        pltpu.VMEM((_EC,), jnp.float32),
        pltpu.VMEM((640,), jnp.float32),
        pltpu.VMEM_SHARED((_NP,), jnp.float32),
        pltpu.SemaphoreType.DMA,
        pltpu.SemaphoreType.DMA,
    ]

    @functools.partial(
        pl.kernel, mesh=mesh,
        out_type=jax.ShapeDtypeStruct((2, _NP), jnp.float32),
        scratch_types=scratch,
    )
    def k(dst2, out, dst_v, one_v, zb_v, denf, sem, sem2):
        c = lax.axis_index("c")
        s = lax.axis_index("s")
        pltpu.sync_copy(dst2.at[pl.ds(s * _ECR, _ECR)], dst_v)
        z16 = jnp.zeros((16,), jnp.float32)
        o16 = jnp.ones((16,), jnp.float32)
        for u in range(5):
            one_v[pl.ds(16 * u, 16)] = o16

        def zstep(j, carry):
            zb_v[pl.ds(j * 16, 16)] = z16
            return carry

        lax.fori_loop(0, 40, zstep, 0)
        pltpu.sync_copy(zb_v, denf.at[pl.ds(s * 640, 640)])
        plsc.subcore_barrier()

        def step(j, carry):
            pltpu.sync_copy(one_v, denf.at[dst_v.at[j]], add=True)
            return carry

        lax.fori_loop(0, _ECR, step, 0)
        plsc.subcore_barrier()
        pltpu.sync_copy(denf.at[pl.ds(s * 640, 640)],
                        out.at[c, pl.ds(s * 640, 640)])

    return k


@functools.lru_cache(maxsize=None)
def _sc_alpha_k():
    return _make_sc_alpha()


@functools.lru_cache(maxsize=None)
def _sc_count_k():
    return _make_sc_count()


def _max_tail(att, amax_ref):
    row = pl.program_id(0) * _BK + lax.broadcasted_iota(jnp.int32, (_BK, 1), 0)
    att = jnp.where(row < _N, att, -jnp.inf)
    m = jnp.max(att, axis=0, keepdims=True)

    @pl.when(pl.program_id(0) == 0)
    def _():
        amax_ref[...] = m

    @pl.when(pl.program_id(0) != 0)
    def _():
        amax_ref[...] = jnp.maximum(amax_ref[...], m)


def _pre_body(x_ref, w_ref, b_ref, dinv_ref, va_ref,
              xn_ref, xs_ref, att_ref, amax_ref):
    x = x_ref[...]
    y = jnp.dot(x, w_ref[...], preferred_element_type=jnp.float32) + b_ref[...]
    xn = jnp.maximum(y, 0.0)
    xn_ref[...] = xn
    xs_ref[...] = xn * dinv_ref[...]
    att = jnp.dot(xn, va_ref[...], preferred_element_type=jnp.float32)
    att_ref[...] = att
    _max_tail(att, amax_ref)


def _node_body(aggg_ref, xs_ref, agga_ref, dinv_ref,
               wg_ref, bg_ref, wa_ref, ba_ref, msk_ref, bsk_ref, va_ref,
               xn_ref, xsn_ref, att_ref, amax_ref):
    gg = aggg_ref[...]
    z = jnp.concatenate([gg[0], gg[1]], axis=1) + xs_ref[...]
    t1 = jnp.dot(z, wg_ref[...], preferred_element_type=jnp.float32)
    t1 = t1 * dinv_ref[...] + bg_ref[...]
    ga = agga_ref[...]
    a = jnp.concatenate([ga[0], ga[1]], axis=1)
    t2 = jnp.dot(a, wa_ref[...], preferred_element_type=jnp.float32) + ba_ref[...]
    o = t1 + t2
    xn = jnp.dot(o, msk_ref[...], preferred_element_type=jnp.float32) + bsk_ref[...]
    xn = jnp.maximum(xn, 0.0)
    xn_ref[...] = xn
    xsn_ref[...] = xn * dinv_ref[...]
    att = jnp.dot(xn, va_ref[...], preferred_element_type=jnp.float32)
    att_ref[...] = att
    _max_tail(att, amax_ref)


def _pool_body(x1_ref, b1_ref, x2_ref, b2_ref,
               wp1_ref, bp1_ref, wp2_ref, bp2_ref,
               p1_ref, c1_ref, p2_ref, c2_ref):
    ids = lax.broadcasted_iota(jnp.int32, (1, _NG), 1)

    def one(x_ref, b_ref, wp_ref, bp_ref, p_ref, c_ref):
        xp = jnp.dot(x_ref[...], wp_ref[...],
                     preferred_element_type=jnp.float32) + bp_ref[...]
        oh = (b_ref[...] == ids).astype(jnp.float32)
        ppart = lax.dot_general(oh, xp, (((0,), (0,)), ((), ())),
                                preferred_element_type=jnp.float32)
        cpart = jnp.sum(oh, axis=0, keepdims=True)

        @pl.when(pl.program_id(0) == 0)
        def _():
            p_ref[...] = ppart
            c_ref[...] = cpart

        @pl.when(pl.program_id(0) != 0)
        def _():
            p_ref[...] += ppart
            c_ref[...] += cpart

    one(x1_ref, b1_ref, wp1_ref, bp1_ref, p1_ref, c1_ref)
    one(x2_ref, b2_ref, wp2_ref, bp2_ref, p2_ref, c2_ref)


def _head_body(p1_ref, c1_ref, p2_ref, c2_ref,
               w1_ref, b1_ref, w2_ref, b2_ref, out_ref):
    c1 = jnp.maximum(c1_ref[...], 1.0)
    c2 = jnp.maximum(c2_ref[...], 1.0)
    r1 = p1_ref[...] * (1.0 / c1.T)
    r2 = p2_ref[...] * (1.0 / c2.T)
    r = jnp.concatenate([r1, r2], axis=1)
    h = jnp.dot(r, w1_ref[...], preferred_element_type=jnp.float32) + b1_ref[...]
    h = jnp.maximum(h, 0.0)
    out_ref[...] = jnp.dot(h, w2_ref[...],
                           preferred_element_type=jnp.float32) + b2_ref[...]


def _full(shape):
    return pl.BlockSpec(shape, lambda i: (0,) * len(shape))


def _rows(shape):
    return pl.BlockSpec(shape, lambda i: (i,) + (0,) * (len(shape) - 1))


def _pre_call(x, w, b, dinv, va):
    return pl.pallas_call(
        _pre_body,
        grid=(_NBLK,),
        in_specs=[_rows((_BK, _H)), _full((_H, _H)), _full((1, _H)),
                  _rows((_BK, 1)), _full((_H, 8))],
        out_specs=[_rows((_BK, _H)), _rows((_BK, _H)), _rows((_BK, 8)),
                   _full((1, 8))],
        out_shape=[jax.ShapeDtypeStruct((_NP, _H), jnp.float32),
                   jax.ShapeDtypeStruct((_NP, _H), jnp.float32),
                   jax.ShapeDtypeStruct((_NP, 8), jnp.float32),
                   jax.ShapeDtypeStruct((1, 8), jnp.float32)],
    )(x, w, b, dinv, va)


def _node_call(aggg, xs, agga, dinv, wg, bg, wa, ba, msk, bsk, va):
    return pl.pallas_call(
        _node_body,
        grid=(_NBLK,),
        in_specs=[
            pl.BlockSpec((2, _BK, _HH), lambda i: (0, i, 0)),
            _rows((_BK, _H)),
            pl.BlockSpec((2, _BK, _HH), lambda i: (0, i, 0)),
            _rows((_BK, 1)),
            _full((_H, _H)), _full((1, _H)),
            _full((_H, _H)), _full((1, _H)),
            _full((_H, _H)), _full((1, _H)),
            _full((_H, 8)),
        ],
        out_specs=[_rows((_BK, _H)), _rows((_BK, _H)), _rows((_BK, 8)),
                   _full((1, 8))],
        out_shape=[jax.ShapeDtypeStruct((_NP, _H), jnp.float32),
                   jax.ShapeDtypeStruct((_NP, _H), jnp.float32),
                   jax.ShapeDtypeStruct((_NP, 8), jnp.float32),
                   jax.ShapeDtypeStruct((1, 8), jnp.float32)],
    )(aggg, xs, agga, dinv, wg, bg, wa, ba, msk, bsk, va)


def _pool_call(x1, b1, x2, b2, wp1, bp1, wp2, bp2):
    return pl.pallas_call(
        _pool_body,
        grid=(_NBLK,),
        in_specs=[_rows((_BK, _H)), _rows((_BK, 1)),
                  _rows((_BK, _H)), _rows((_BK, 1)),
                  _full((_H, _H)), _full((1, _H)),
                  _full((_H, _H)), _full((1, _H))],
        out_specs=[_full((_NG, _H)), _full((1, _NG)),
                   _full((_NG, _H)), _full((1, _NG))],
        out_shape=[jax.ShapeDtypeStruct((_NG, _H), jnp.float32),
                   jax.ShapeDtypeStruct((1, _NG), jnp.float32),
                   jax.ShapeDtypeStruct((_NG, _H), jnp.float32),
                   jax.ShapeDtypeStruct((1, _NG), jnp.float32)],
    )(x1, b1, x2, b2, wp1, bp1, wp2, bp2)


def _head_call(p1, c1, p2, c2, w1, b1, w2, b2):
    return pl.pallas_call(
        _head_body,
        grid=(1,),
        in_specs=[_full((_NG, _H)), _full((1, _NG)),
                  _full((_NG, _H)), _full((1, _NG)),
                  _full((2 * _H, _H)), _full((1, _H)),
                  _full((_H, 32)), _full((1, 32))],
        out_specs=_full((_NG, 32)),
        out_shape=jax.ShapeDtypeStruct((_NG, 32), jnp.float32),
    )(p1, c1, p2, c2, w1, b1, w2, b2)


def _fold_bn(W, b, g, beta, m, v):
    inv = g * lax.rsqrt(v + 1e-5)
    return (W.T * inv[None, :], ((b - m) * inv + beta)[None, :])


def _lrelu(x):
    return jnp.maximum(x, 0.2 * x)


def _seg_sum_rows(vals, src, dst, coeff, n_dst):
    g = vals[src]
    if coeff is not None:
        g = g * coeff[:, None]
    return jnp.zeros((n_dst, vals.shape[1]), vals.dtype).at[dst].add(g)


def _edge_alpha(als, ald, src, dst, mshift, n_dst):
    e = _lrelu(als[src] + ald[dst])
    ex = jnp.exp(e - mshift)
    den = jnp.zeros((n_dst,), ex.dtype).at[dst].add(ex)
    return ex / jnp.maximum(den[dst], 1e-16)


def kernel(x_graph_1, x_graph_2, edge_index_g1_g1, edge_index_g2_g2,
           edge_index_g1_g2, edge_index_g2_g1, batch_graph_1, batch_graph_2,
           slice_dict, params):
    p = params
    L = 2

    wpre1, bpre1 = _fold_bn(p['pre_W_g1'], p['pre_b_g1'],
                            p['bn_pre_g1_gamma'], p['bn_pre_g1_beta'],
                            p['bn_pre_g1_mean'], p['bn_pre_g1_var'])
    wpre2, bpre2 = _fold_bn(p['pre_W_g2'], p['pre_b_g2'],
                            p['bn_pre_g2_gamma'], p['bn_pre_g2_beta'],
                            p['bn_pre_g2_mean'], p['bn_pre_g2_var'])
    wpost1, bpost1 = _fold_bn(p['post_W_g1'], p['post_b_g1'],
                              p['bn_post_g1_gamma'], p['bn_post_g1_beta'],
                              p['bn_post_g1_mean'], p['bn_post_g1_var'])
    wpost2, bpost2 = _fold_bn(p['post_W_g2'], p['post_b_g2'],
                              p['bn_post_g2_gamma'], p['bn_post_g2_beta'],
                              p['bn_post_g2_mean'], p['bn_post_g2_var'])

    eye = jnp.eye(_H, dtype=jnp.float32)
    zcol = jnp.zeros((_H, 1), jnp.float32)

    def att_w(l):
        v_as12 = p['gat12_Ws_%d' % l].T @ p['gat12_as_%d' % l]
        v_ad12 = p['gat12_Wd_%d' % l].T @ p['gat12_ad_%d' % l]
        v_as21 = p['gat21_Ws_%d' % l].T @ p['gat21_as_%d' % l]
        v_ad21 = p['gat21_Wd_%d' % l].T @ p['gat21_ad_%d' % l]
        va1 = jnp.concatenate([v_as12[:, None], v_ad21[:, None]] + [zcol] * 6,
                              axis=1)
        va2 = jnp.concatenate([v_as21[:, None], v_ad12[:, None]] + [zcol] * 6,
                              axis=1)
        return va1, va2

    va_zero = jnp.zeros((_H, 8), jnp.float32)

    src11, dst11 = edge_index_g1_g1[0], edge_index_g1_g1[1]
    src22, dst22 = edge_index_g2_g2[0], edge_index_g2_g2[1]
    src12, dst12 = edge_index_g1_g2[0], edge_index_g1_g2[1]
    src21, dst21 = edge_index_g2_g1[0], edge_index_g2_g1[1]

    zrows = jnp.zeros((_NP, _HH), jnp.float32)
    epad = _E2D * _EC - _E
    spad = jnp.zeros((epad,), jnp.int32)
    dpad = jnp.full((epad,), _N, jnp.int32)

    def e2d(v, pad):
        return jnp.concatenate([v, pad]).reshape(_E2D, _EC)

    s11_2d, d11_2d = e2d(src11, spad), e2d(dst11, dpad)
    s22_2d, d22_2d = e2d(src22, spad), e2d(dst22, dpad)
    s12_2d, d12_2d = e2d(src12, spad), e2d(dst12, dpad)
    s21_2d, d21_2d = e2d(src21, spad), e2d(dst21, dpad)

    deg1 = _sc_count_k()(d11_2d)[0] + 1.0
    deg2 = _sc_count_k()(d22_2d)[0] + 1.0
    dinv1 = lax.rsqrt(deg1)[:, None]
    dinv2 = lax.rsqrt(deg2)[:, None]

    npad = _NP - _N
    x1in = jnp.pad(x_graph_1, ((0, npad), (0, 0)))
    x2in = jnp.pad(x_graph_2, ((0, npad), (0, 0)))
    va1, va2 = att_w(0)
    x1, xs1, att1, am1 = _pre_call(x1in, wpre1, bpre1, dinv1, va1)
    x2, xs2, att2, am2 = _pre_call(x2in, wpre2, bpre2, dinv2, va2)

    for l in range(L):
        aggg1 = _sc_agg(False)(xs1[:, :_HH], xs1[:, _HH:], s11_2d, d11_2d,
                               zrows)
        aggg2 = _sc_agg(False)(xs2[:, :_HH], xs2[:, _HH:], s22_2d, d22_2d,
                               zrows)

        m12 = _lrelu(am1[0, 0] + am2[0, 1])
        m21 = _lrelu(am2[0, 0] + am1[0, 1])
        al12 = _sc_alpha_k()(att1[:, 0], att2[:, 1], s12_2d, d12_2d,
                             jnp.full((16,), m12, jnp.float32))
        al21 = _sc_alpha_k()(att2[:, 0], att1[:, 1], s21_2d, d21_2d,
                             jnp.full((16,), m21, jnp.float32))
        agga1 = _sc_agg(True)(x2[:, :_HH], x2[:, _HH:], s21_2d, d21_2d,
                              al21, zrows)
        agga2 = _sc_agg(True)(x1[:, :_HH], x1[:, _HH:], s12_2d, d12_2d,
                              al12, zrows)

        van1, van2 = att_w(l + 1) if l + 1 < L else (va_zero, va_zero)
        x1, xs1, att1, am1 = _node_call(
            aggg1, xs1, agga1, dinv1,
            p['gcn11_W_%d' % l].T, p['gcn11_b_%d' % l][None, :],
            p['gat21_Ws_%d' % l].T, p['gat21_b_%d' % l][None, :],
            eye + p['skip_W_g1_%d' % l].T, p['skip_b_g1_%d' % l][None, :],
            van1)
        x2, xs2, att2, am2 = _node_call(
            aggg2, xs2, agga2, dinv2,
            p['gcn22_W_%d' % l].T, p['gcn22_b_%d' % l][None, :],
            p['gat12_Ws_%d' % l].T, p['gat12_b_%d' % l][None, :],
            eye + p['skip_W_g2_%d' % l].T, p['skip_b_g2_%d' % l][None, :],
            van2)

    b1p = jnp.pad(batch_graph_1, (0, npad), constant_values=_NG)[:, None]
    b2p = jnp.pad(batch_graph_2, (0, npad), constant_values=_NG)[:, None]
    p1, c1, p2, c2 = _pool_call(x1, b1p, x2, b2p,
                                wpost1, bpost1, wpost2, bpost2)
    return _head_call(p1, c1, p2, c2,
                      p['lin1_W'].T, p['lin1_b'][None, :],
                      p['lin2_W'].T, p['lin2_b'][None, :])
